# baseline (speedup 1.0000x reference)
import functools

import jax
import jax.numpy as jnp
from jax import lax
from jax.experimental import pallas as pl
from jax.experimental.pallas import tpu as pltpu
from jax.experimental.pallas import tpu_sc as plsc

T = 2048
D = 768
FF = 3072
G = 3
E = 8
WPAD = 128
BT = 256
NB = T // BT
TP = 2560


def _route_body(x_ref, wg_ref, bg_ref, wc_ref, bc_ref, wr_ref, br_ref,
                w_ref, rank_ref, perm_ref, offs_ref, meta_ref):
    x = x_ref[...]
    glog = jnp.dot(x, wg_ref[...]) + bg_ref[...]
    m = jnp.max(glog, axis=1, keepdims=True)
    eq = (glog >= m).astype(jnp.float32)
    e0 = eq[:, 0:1]
    e1 = eq[:, 1:2]
    e2 = eq[:, 2:3]
    oh = jnp.concatenate(
        [e0, e1 * (1.0 - e0), e2 * (1.0 - jnp.maximum(e0, e1))], axis=1)

    pooled = jnp.mean(x, axis=0, keepdims=True)
    cvec = jnp.dot(pooled, wc_ref[D:2 * D, :]) + bc_ref[...]
    cond = jnp.dot(x, wc_ref[0:D, :]) + cvec
    el = jnp.dot(cond, wr_ref[...]) + br_ref[...]

    w = jnp.zeros((T, E), dtype=jnp.float32)
    for g in range(G):
        sg = el[:, g * E:(g + 1) * E]
        mg = jnp.max(sg, axis=1, keepdims=True)
        ex = jnp.exp(sg - mg)
        pg = ex / jnp.sum(ex, axis=1, keepdims=True)
        w = w + oh[:, g:g + 1] * pg
    w_ref[:, 0:E] = w
    w_ref[:, E:WPAD] = jnp.zeros((T, WPAD - E), dtype=jnp.float32)

    acc = oh
    shift = 1
    while shift < T:
        acc = acc + jnp.concatenate(
            [jnp.zeros((shift, G), dtype=jnp.float32), acc[:T - shift, :]],
            axis=0)
        shift *= 2
    cs_excl = acc - oh
    counts = acc[T - 1:T, :]
    offe = jnp.concatenate(
        [jnp.zeros((1, 1), dtype=jnp.float32),
         counts[:, 0:1],
         counts[:, 0:1] + counts[:, 1:2]], axis=1)
    rank_f = jnp.sum(oh * (cs_excl + offe), axis=1, keepdims=True)
    rank_i = rank_f.astype(jnp.int32)
    rank_ref[...] = rank_i

    t_col = lax.broadcasted_iota(
        jnp.int32, (T, 1), 0).astype(jnp.float32)
    for ci in range(T // 256):
        icol = 256 * ci + lax.broadcasted_iota(jnp.int32, (1, 256), 1)
        a = (rank_i == icol).astype(jnp.float32)
        perm_ref[ci:ci + 1, :] = jnp.sum(
            a * t_col, axis=0, keepdims=True).astype(jnp.int32)

    offlo = offe
    offhi = jnp.concatenate(
        [offe[:, 1:G], jnp.full((1, 1), float(T), jnp.float32)], axis=1)
    ne = (counts > 0.0).astype(jnp.float32)
    bf = jnp.floor(offlo / float(BT))
    bl = jnp.floor((offhi - 1.0) / float(BT))
    bc = (bl - bf + 1.0) * ne
    z1 = jnp.zeros((1, 1), jnp.float32)
    a1 = bc[:, 0:1]
    a2 = a1 + bc[:, 1:2]
    a3 = a2 + bc[:, 2:3]
    bcum3 = jnp.concatenate([z1, a1, a2], axis=1)
    bcumhi = jnp.concatenate([a1, a2, a3], axis=1)
    ns0 = ne[:, 0:1]
    nsum = jnp.concatenate([z1, ns0, ns0 + ne[:, 1:2]], axis=1)
    S = ns0 + ne[:, 1:2] + ne[:, 2:3]
    jio = lax.broadcasted_iota(jnp.int32, (16, 1), 0).astype(jnp.float32)
    gh = ((jio >= bcum3) & (jio < bcumhi)).astype(jnp.float32)
    gidxr = lax.broadcasted_iota(jnp.int32, (1, G), 1).astype(jnp.float32)
    validj = jnp.sum(gh, axis=1, keepdims=True)
    gj = jnp.sum(gh * gidxr, axis=1, keepdims=True)
    bj = jnp.sum(gh * (bf - bcum3), axis=1, keepdims=True) + jio * validj
    segj = jnp.sum(gh * nsum, axis=1, keepdims=True)
    lastmask = (jio == (a3 - 1.0)).astype(jnp.float32)
    lastg = jnp.sum(lastmask * gj, axis=0, keepdims=True)
    lastb = jnp.sum(lastmask * bj, axis=0, keepdims=True)
    lastseg = jnp.sum(lastmask * segj, axis=0, keepdims=True)
    inv = 1.0 - validj
    gj = gj * validj + lastg * inv
    bj = bj * validj + lastb * inv
    segj = segj * validj + lastseg * inv
    segprev = jnp.concatenate([segj[0:1, :], segj[0:15, :]], axis=0)
    news = ((jio == 0.0) | (segj != segprev)).astype(jnp.float32)
    nlast = (segj == (S - 1.0)).astype(jnp.float32)
    stgt = (segj + 1.0) * (1.0 - nlast)
    ngj = jnp.sum(((nsum == stgt).astype(jnp.float32) * ne) * gidxr,
                  axis=1, keepdims=True)
    meta = jnp.concatenate(
        [gj, bj, validj, segj, news, ngj, nlast,
         jnp.zeros((16, 1), jnp.float32)], axis=1)
    meta_ref[...] = meta.astype(jnp.int32)

    offs = jnp.concatenate([offe, S, offhi, z1], axis=1)
    offs_ref[...] = offs.astype(jnp.int32)


def _route(x, Wg, bg, Wc, bc, Wr2, br2):
    return pl.pallas_call(
        _route_body,
        out_shape=[
            jax.ShapeDtypeStruct((T, WPAD), jnp.float32),
            jax.ShapeDtypeStruct((T, 1), jnp.int32),
            jax.ShapeDtypeStruct((T // 256, 256), jnp.int32),
            jax.ShapeDtypeStruct((1, 8), jnp.int32),
            jax.ShapeDtypeStruct((16, 8), jnp.int32),
        ],
    )(x, Wg, bg, Wc, bc, Wr2, br2)


@functools.lru_cache(maxsize=None)
def _sc_mesh_and_workers():
    info = plsc.get_sparse_core_info()
    nc, ns = info.num_cores, info.num_subcores
    mesh = plsc.VectorSubcoreMesh(
        core_axis_name="c", subcore_axis_name="s",
        num_cores=nc, num_subcores=ns)
    return mesh, nc, nc * ns


@functools.lru_cache(maxsize=None)
def _gather2_kernel():
    mesh, nc, nw = _sc_mesh_and_workers()
    bpw = T // nw

    @functools.partial(
        pl.kernel,
        out_type=[jax.ShapeDtypeStruct((T, D), jnp.float32),
                  jax.ShapeDtypeStruct((T, WPAD), jnp.float32)],
        mesh=mesh,
        scratch_types=[
            pltpu.VMEM((bpw,), jnp.int32),
            pltpu.VMEM((bpw, D), jnp.float32),
            pltpu.VMEM((bpw, WPAD), jnp.float32),
            pltpu.SemaphoreType.DMA,
        ],
    )
    def k(x_hbm, w_hbm, idx_hbm, ox_hbm, ow_hbm, idx_v, rx_v, rw_v, sem):
        wid = lax.axis_index("s") * nc + lax.axis_index("c")
        base = wid * bpw
        pltpu.sync_copy(idx_hbm.at[pl.ds(base, bpw)], idx_v)
        pltpu.async_copy(x_hbm.at[idx_v], rx_v, sem).wait()
        pltpu.sync_copy(rx_v, ox_hbm.at[pl.ds(base, bpw)])
        pltpu.async_copy(w_hbm.at[idx_v], rw_v, sem).wait()
        pltpu.sync_copy(rw_v, ow_hbm.at[pl.ds(base, bpw)])

    return k


@functools.lru_cache(maxsize=None)
def _gather1_kernel():
    mesh, nc, nw = _sc_mesh_and_workers()
    bpw = T // nw

    @functools.partial(
        pl.kernel,
        out_type=jax.ShapeDtypeStruct((T, D), jnp.float32),
        mesh=mesh,
        scratch_types=[
            pltpu.VMEM((bpw,), jnp.int32),
            pltpu.VMEM((bpw, D), jnp.float32),
            pltpu.SemaphoreType.DMA,
        ],
    )
    def k(ys_hbm, idx_hbm, oy_hbm, idx_v, ry_v, sem):
        wid = lax.axis_index("s") * nc + lax.axis_index("c")
        base = wid * bpw
        pltpu.sync_copy(idx_hbm.at[pl.ds(base, bpw)], idx_v)
        pltpu.async_copy(ys_hbm.at[idx_v], ry_v, sem).wait()
        pltpu.sync_copy(ry_v, oy_hbm.at[pl.ds(base, bpw)])

    return k


JT = NB + G - 1


def _ffn_body(offs_ref, meta_ref, xs_ref, ws_ref, w1_hbm, b1_ref, w2_hbm,
              b2_ref, out_ref, w1buf, w2buf, w1bb, w2bb, sem1, sem2):
    e = pl.program_id(0)
    j = pl.program_id(1)
    gg = meta_ref[j, 0]
    bb = meta_ref[j, 1]
    valid = meta_ref[j, 2]
    sj = meta_ref[j, 3]
    news = meta_ref[j, 4]
    ngj = meta_ref[j, 5]
    nl = meta_ref[j, 6]
    S = offs_ref[3]
    pos = e * S + sj

    @pl.when(news == 1)
    def _stage():
        @pl.when(pos == 0)
        def _prologue():
            g0 = meta_ref[0, 0]
            pltpu.make_async_copy(w1_hbm.at[g0, 0], w1buf, sem1).start()
            pltpu.make_async_copy(w2_hbm.at[g0, 0], w2buf, sem2).start()
        pltpu.make_async_copy(w1_hbm.at[gg, e], w1buf, sem1).wait()
        pltpu.make_async_copy(w2_hbm.at[gg, e], w2buf, sem2).wait()
        w1bb[...] = w1buf[...].astype(jnp.bfloat16)
        w2bb[...] = w2buf[...].astype(jnp.bfloat16)

        @pl.when(pos + 1 < E * S)
        def _next():
            en = e + nl
            pltpu.make_async_copy(w1_hbm.at[ngj, en], w1buf, sem1).start()
            pltpu.make_async_copy(w2_hbm.at[ngj, en], w2buf, sem2).start()

    @pl.when((e == 0) & (j == 0))
    def _init():
        out_ref[...] = xs_ref[...]

    start = offs_ref[gg]
    end = offs_ref[4 + gg]
    row0 = bb * BT

    @pl.when(valid == 1)
    def _work():
        xb = xs_ref[pl.ds(row0, BT), :].astype(jnp.bfloat16)
        onehot = (lax.broadcasted_iota(jnp.int32, (WPAD, 1), 0)
                  == e).astype(jnp.float32)
        wcol = jnp.dot(ws_ref[pl.ds(row0, BT), :], onehot)
        rid = row0 + lax.broadcasted_iota(jnp.int32, (BT, 1), 0)
        inseg = (rid >= start) & (rid < end)
        wm = jnp.where(inseg, wcol, 0.0)
        h = jax.nn.gelu(
            jnp.dot(xb, w1bb[...], preferred_element_type=jnp.float32)
            + b1_ref[0, 0])
        h16 = (wm * h).astype(jnp.bfloat16)
        contrib = jnp.dot(h16, w2bb[...],
                          preferred_element_type=jnp.float32)
        out_ref[pl.ds(row0, BT), :] += contrib + wm * b2_ref[0, 0]


def _ffn(offs, meta, xs, ws, W1, b1r, W2, b2r):
    grid_spec = pltpu.PrefetchScalarGridSpec(
        num_scalar_prefetch=2,
        grid=(E, JT),
        in_specs=[
            pl.BlockSpec((T, D), lambda e, j, offs, meta: (0, 0)),
            pl.BlockSpec((T, WPAD), lambda e, j, offs, meta: (0, 0)),
            pl.BlockSpec(memory_space=pl.ANY),
            pl.BlockSpec((1, 1, 1, FF),
                         lambda e, j, offs, meta: (meta[j, 0], e, 0, 0)),
            pl.BlockSpec(memory_space=pl.ANY),
            pl.BlockSpec((1, 1, 1, D),
                         lambda e, j, offs, meta: (meta[j, 0], e, 0, 0)),
        ],
        out_specs=pl.BlockSpec((T, D), lambda e, j, offs, meta: (0, 0)),
        scratch_shapes=[
            pltpu.VMEM((D, FF), jnp.float32),
            pltpu.VMEM((FF, D), jnp.float32),
            pltpu.VMEM((D, FF), jnp.bfloat16),
            pltpu.VMEM((FF, D), jnp.bfloat16),
            pltpu.SemaphoreType.DMA,
            pltpu.SemaphoreType.DMA,
        ],
    )
    return pl.pallas_call(
        _ffn_body,
        grid_spec=grid_spec,
        out_shape=jax.ShapeDtypeStruct((T, D), jnp.float32),
        compiler_params=pltpu.CompilerParams(
            dimension_semantics=("arbitrary", "arbitrary")),
    )(offs, meta, xs, ws, W1, b1r, W2, b2r)


def kernel(x, Wg, bg, Wc, bc, Wr, br, W1, b1, W2, b2):
    Wr2 = jnp.transpose(Wr, (1, 0, 2)).reshape(D, G * E)
    br2 = br.reshape(1, G * E)
    wpad, rank2, perm2, offs2, meta = _route(
        x, Wg, bg.reshape(1, G), Wc, bc.reshape(1, D), Wr2, br2)
    rank = rank2.reshape(T)
    perm = perm2.reshape(T)
    offs = offs2.reshape(8)

    xs, ws = _gather2_kernel()(x, wpad, perm)
    ys = _ffn(offs, meta, xs, ws,
              W1, b1.reshape(G, E, 1, FF), W2, b2.reshape(G, E, 1, D))
    return _gather1_kernel()(ys, rank)


# speedup vs baseline: 1.0810x; 1.0810x over previous
import functools

import jax
import jax.numpy as jnp
from jax import lax
from jax.experimental import pallas as pl
from jax.experimental.pallas import tpu as pltpu
from jax.experimental.pallas import tpu_sc as plsc

T = 2048
D = 768
FF = 3072
G = 3
E = 8
WPAD = 128
BT = 256
NB = T // BT
TP = 2560


def _route_body(x_ref, wg_ref, bg_ref, wc_ref, bc_ref, wr_ref, br_ref,
                w_ref, rank_ref, perm_ref, offs_ref, meta_ref):
    x = x_ref[...]
    glog = jnp.dot(x, wg_ref[...]) + bg_ref[...]
    m = jnp.max(glog, axis=1, keepdims=True)
    eq = (glog >= m).astype(jnp.float32)
    e0 = eq[:, 0:1]
    e1 = eq[:, 1:2]
    e2 = eq[:, 2:3]
    oh = jnp.concatenate(
        [e0, e1 * (1.0 - e0), e2 * (1.0 - jnp.maximum(e0, e1))], axis=1)

    pooled = jnp.mean(x, axis=0, keepdims=True)
    cvec = jnp.dot(pooled, wc_ref[D:2 * D, :]) + bc_ref[...]
    cond = jnp.dot(x, wc_ref[0:D, :]) + cvec
    el = jnp.dot(cond, wr_ref[...]) + br_ref[...]

    sel = (oh[:, 0:1] * el[:, 0:E] + oh[:, 1:2] * el[:, E:2 * E]
           + oh[:, 2:3] * el[:, 2 * E:3 * E])
    mg = jnp.max(sel, axis=1, keepdims=True)
    ex = jnp.exp(sel - mg)
    w = ex / jnp.sum(ex, axis=1, keepdims=True)
    w_ref[:, 0:E] = w
    w_ref[:, E:WPAD] = jnp.zeros((T, WPAD - E), dtype=jnp.float32)

    acc = oh
    shift = 1
    while shift < T:
        acc = acc + jnp.concatenate(
            [jnp.zeros((shift, G), dtype=jnp.float32), acc[:T - shift, :]],
            axis=0)
        shift *= 2
    cs_excl = acc - oh
    counts = acc[T - 1:T, :]
    offe = jnp.concatenate(
        [jnp.zeros((1, 1), dtype=jnp.float32),
         counts[:, 0:1],
         counts[:, 0:1] + counts[:, 1:2]], axis=1)
    rank_f = jnp.sum(oh * (cs_excl + offe), axis=1, keepdims=True)
    rank_i = rank_f.astype(jnp.int32)
    rank_ref[...] = rank_i

    t_col = lax.broadcasted_iota(
        jnp.int32, (T, 1), 0).astype(jnp.float32)
    for ci in range(T // 256):
        icol = 256 * ci + lax.broadcasted_iota(jnp.int32, (1, 256), 1)
        a = (rank_i == icol).astype(jnp.float32)
        perm_ref[ci:ci + 1, :] = jnp.sum(
            a * t_col, axis=0, keepdims=True).astype(jnp.int32)

    offlo = offe
    offhi = jnp.concatenate(
        [offe[:, 1:G], jnp.full((1, 1), float(T), jnp.float32)], axis=1)
    ne = (counts > 0.0).astype(jnp.float32)
    bf = jnp.floor(offlo / float(BT))
    bl = jnp.floor((offhi - 1.0) / float(BT))
    bc = (bl - bf + 1.0) * ne
    z1 = jnp.zeros((1, 1), jnp.float32)
    a1 = bc[:, 0:1]
    a2 = a1 + bc[:, 1:2]
    a3 = a2 + bc[:, 2:3]
    bcum3 = jnp.concatenate([z1, a1, a2], axis=1)
    bcumhi = jnp.concatenate([a1, a2, a3], axis=1)
    ns0 = ne[:, 0:1]
    nsum = jnp.concatenate([z1, ns0, ns0 + ne[:, 1:2]], axis=1)
    S = ns0 + ne[:, 1:2] + ne[:, 2:3]
    jio = lax.broadcasted_iota(jnp.int32, (16, 1), 0).astype(jnp.float32)
    gh = ((jio >= bcum3) & (jio < bcumhi)).astype(jnp.float32)
    gidxr = lax.broadcasted_iota(jnp.int32, (1, G), 1).astype(jnp.float32)
    validj = jnp.sum(gh, axis=1, keepdims=True)
    gj = jnp.sum(gh * gidxr, axis=1, keepdims=True)
    bj = jnp.sum(gh * (bf - bcum3), axis=1, keepdims=True) + jio * validj
    segj = jnp.sum(gh * nsum, axis=1, keepdims=True)
    lastmask = (jio == (a3 - 1.0)).astype(jnp.float32)
    lastg = jnp.sum(lastmask * gj, axis=0, keepdims=True)
    lastb = jnp.sum(lastmask * bj, axis=0, keepdims=True)
    lastseg = jnp.sum(lastmask * segj, axis=0, keepdims=True)
    inv = 1.0 - validj
    gj = gj * validj + lastg * inv
    bj = bj * validj + lastb * inv
    segj = segj * validj + lastseg * inv
    segprev = jnp.concatenate([segj[0:1, :], segj[0:15, :]], axis=0)
    news = ((jio == 0.0) | (segj != segprev)).astype(jnp.float32)
    nlast = (segj == (S - 1.0)).astype(jnp.float32)
    stgt = (segj + 1.0) * (1.0 - nlast)
    ngj = jnp.sum(((nsum == stgt).astype(jnp.float32) * ne) * gidxr,
                  axis=1, keepdims=True)
    meta = jnp.concatenate(
        [gj, bj, validj, segj, news, ngj, nlast,
         jnp.zeros((16, 1), jnp.float32)], axis=1)
    meta_ref[...] = meta.astype(jnp.int32)

    offs = jnp.concatenate([offe, S, offhi, z1], axis=1)
    offs_ref[...] = offs.astype(jnp.int32)


def _route(x, Wg, bg, Wc, bc, Wr2, br2):
    return pl.pallas_call(
        _route_body,
        out_shape=[
            jax.ShapeDtypeStruct((T, WPAD), jnp.float32),
            jax.ShapeDtypeStruct((T, 1), jnp.int32),
            jax.ShapeDtypeStruct((T // 256, 256), jnp.int32),
            jax.ShapeDtypeStruct((1, 8), jnp.int32),
            jax.ShapeDtypeStruct((16, 8), jnp.int32),
        ],
    )(x, Wg, bg, Wc, bc, Wr2, br2)


@functools.lru_cache(maxsize=None)
def _sc_mesh_and_workers():
    info = plsc.get_sparse_core_info()
    nc, ns = info.num_cores, info.num_subcores
    mesh = plsc.VectorSubcoreMesh(
        core_axis_name="c", subcore_axis_name="s",
        num_cores=nc, num_subcores=ns)
    return mesh, nc, nc * ns


@functools.lru_cache(maxsize=None)
def _gather2_kernel():
    mesh, nc, nw = _sc_mesh_and_workers()
    bpw = T // nw

    @functools.partial(
        pl.kernel,
        out_type=[jax.ShapeDtypeStruct((T, D), jnp.float32),
                  jax.ShapeDtypeStruct((T, WPAD), jnp.float32)],
        mesh=mesh,
        scratch_types=[
            pltpu.VMEM((bpw,), jnp.int32),
            pltpu.VMEM((bpw, D), jnp.float32),
            pltpu.VMEM((bpw, WPAD), jnp.float32),
            pltpu.SemaphoreType.DMA,
        ],
    )
    def k(x_hbm, w_hbm, idx_hbm, ox_hbm, ow_hbm, idx_v, rx_v, rw_v, sem):
        wid = lax.axis_index("s") * nc + lax.axis_index("c")
        base = wid * bpw
        pltpu.sync_copy(idx_hbm.at[pl.ds(base, bpw)], idx_v)
        pltpu.async_copy(x_hbm.at[idx_v], rx_v, sem).wait()
        pltpu.sync_copy(rx_v, ox_hbm.at[pl.ds(base, bpw)])
        pltpu.async_copy(w_hbm.at[idx_v], rw_v, sem).wait()
        pltpu.sync_copy(rw_v, ow_hbm.at[pl.ds(base, bpw)])

    return k


@functools.lru_cache(maxsize=None)
def _gather1_kernel():
    mesh, nc, nw = _sc_mesh_and_workers()
    bpw = T // nw

    @functools.partial(
        pl.kernel,
        out_type=jax.ShapeDtypeStruct((T, D), jnp.float32),
        mesh=mesh,
        scratch_types=[
            pltpu.VMEM((bpw,), jnp.int32),
            pltpu.VMEM((bpw, D), jnp.float32),
            pltpu.SemaphoreType.DMA,
        ],
    )
    def k(ys_hbm, idx_hbm, oy_hbm, idx_v, ry_v, sem):
        wid = lax.axis_index("s") * nc + lax.axis_index("c")
        base = wid * bpw
        pltpu.sync_copy(idx_hbm.at[pl.ds(base, bpw)], idx_v)
        pltpu.async_copy(ys_hbm.at[idx_v], ry_v, sem).wait()
        pltpu.sync_copy(ry_v, oy_hbm.at[pl.ds(base, bpw)])

    return k


JT = NB + G - 1


def _ffn_body(offs_ref, meta_ref, xs_ref, ws_ref, w1_hbm, b1_ref, w2_hbm,
              b2_ref, out_ref, w1buf, w2buf, sem1, sem2):
    e = pl.program_id(0)
    j = pl.program_id(1)
    gg = meta_ref[j, 0]
    bb = meta_ref[j, 1]
    valid = meta_ref[j, 2]
    sj = meta_ref[j, 3]
    news = meta_ref[j, 4]
    ngj = meta_ref[j, 5]
    nl = meta_ref[j, 6]
    S = offs_ref[3]
    pos = e * S + sj
    p = lax.rem(pos, 2)

    @pl.when(news == 1)
    def _stage():
        @pl.when(pos == 0)
        def _prologue():
            g0 = meta_ref[0, 0]
            pltpu.make_async_copy(w1_hbm.at[g0, 0], w1buf.at[0],
                                  sem1.at[0]).start()
            pltpu.make_async_copy(w2_hbm.at[g0, 0], w2buf.at[0],
                                  sem2.at[0]).start()
        pltpu.make_async_copy(w1_hbm.at[gg, e], w1buf.at[p], sem1.at[p]).wait()
        pltpu.make_async_copy(w2_hbm.at[gg, e], w2buf.at[p], sem2.at[p]).wait()

        @pl.when(pos + 1 < E * S)
        def _next():
            en = e + nl
            pn = lax.rem(pos + 1, 2)
            pltpu.make_async_copy(w1_hbm.at[ngj, en], w1buf.at[pn],
                                  sem1.at[pn]).start()
            pltpu.make_async_copy(w2_hbm.at[ngj, en], w2buf.at[pn],
                                  sem2.at[pn]).start()

    @pl.when((e == 0) & (j == 0))
    def _init():
        out_ref[...] = xs_ref[...]

    start = offs_ref[gg]
    end = offs_ref[4 + gg]
    row0 = bb * BT

    @pl.when(valid == 1)
    def _work():
        xb = xs_ref[pl.ds(row0, BT), :]
        onehot = (lax.broadcasted_iota(jnp.int32, (WPAD, 1), 0)
                  == e).astype(jnp.float32)
        wcol = jnp.dot(ws_ref[pl.ds(row0, BT), :], onehot)
        rid = row0 + lax.broadcasted_iota(jnp.int32, (BT, 1), 0)
        inseg = (rid >= start) & (rid < end)
        wm = jnp.where(inseg, wcol, 0.0)
        h = jax.nn.gelu(jnp.dot(xb, w1buf[p]) + b1_ref[0, 0])
        contrib = jnp.dot(wm * h, w2buf[p])
        out_ref[pl.ds(row0, BT), :] += contrib + wm * b2_ref[0, 0]


def _ffn(offs, meta, xs, ws, W1, b1r, W2, b2r):
    grid_spec = pltpu.PrefetchScalarGridSpec(
        num_scalar_prefetch=2,
        grid=(E, JT),
        in_specs=[
            pl.BlockSpec((T, D), lambda e, j, offs, meta: (0, 0)),
            pl.BlockSpec((T, WPAD), lambda e, j, offs, meta: (0, 0)),
            pl.BlockSpec(memory_space=pl.ANY),
            pl.BlockSpec((1, 1, 1, FF),
                         lambda e, j, offs, meta: (meta[j, 0], e, 0, 0)),
            pl.BlockSpec(memory_space=pl.ANY),
            pl.BlockSpec((1, 1, 1, D),
                         lambda e, j, offs, meta: (meta[j, 0], e, 0, 0)),
        ],
        out_specs=pl.BlockSpec((T, D), lambda e, j, offs, meta: (0, 0)),
        scratch_shapes=[
            pltpu.VMEM((2, D, FF), jnp.float32),
            pltpu.VMEM((2, FF, D), jnp.float32),
            pltpu.SemaphoreType.DMA((2,)),
            pltpu.SemaphoreType.DMA((2,)),
        ],
    )
    return pl.pallas_call(
        _ffn_body,
        grid_spec=grid_spec,
        out_shape=jax.ShapeDtypeStruct((T, D), jnp.float32),
        compiler_params=pltpu.CompilerParams(
            dimension_semantics=("arbitrary", "arbitrary")),
    )(offs, meta, xs, ws, W1, b1r, W2, b2r)


def kernel(x, Wg, bg, Wc, bc, Wr, br, W1, b1, W2, b2):
    Wr2 = jnp.transpose(Wr, (1, 0, 2)).reshape(D, G * E)
    br2 = br.reshape(1, G * E)
    wpad, rank2, perm2, offs2, meta = _route(
        x, Wg, bg.reshape(1, G), Wc, bc.reshape(1, D), Wr2, br2)
    rank = rank2.reshape(T)
    perm = perm2.reshape(T)
    offs = offs2.reshape(8)

    xs, ws = _gather2_kernel()(x, wpad, perm)
    ys = _ffn(offs, meta, xs, ws,
              W1, b1.reshape(G, E, 1, FF), W2, b2.reshape(G, E, 1, D))
    return _gather1_kernel()(ys, rank)


# speedup vs baseline: 1.0840x; 1.0027x over previous
import functools

import jax
import jax.numpy as jnp
from jax import lax
from jax.experimental import pallas as pl
from jax.experimental.pallas import tpu as pltpu
from jax.experimental.pallas import tpu_sc as plsc

T = 2048
D = 768
FF = 3072
G = 3
E = 8
WPAD = 128
BT = 256
NB = T // BT
TP = 2560


def _route_body(x_ref, wg_ref, bg_ref, wc_ref, bc_ref, wr_ref, br_ref,
                w_ref, rank_ref, perm_ref, offs_ref, meta_ref):
    x = x_ref[...]
    glog = jnp.dot(x, wg_ref[...]) + bg_ref[...]
    m = jnp.max(glog, axis=1, keepdims=True)
    eq = (glog >= m).astype(jnp.float32)
    e0 = eq[:, 0:1]
    e1 = eq[:, 1:2]
    e2 = eq[:, 2:3]
    oh = jnp.concatenate(
        [e0, e1 * (1.0 - e0), e2 * (1.0 - jnp.maximum(e0, e1))], axis=1)

    pooled = jnp.mean(x, axis=0, keepdims=True)
    cvec = jnp.dot(pooled, wc_ref[D:2 * D, :]) + bc_ref[...]
    cond = jnp.dot(x, wc_ref[0:D, :]) + cvec
    el = jnp.dot(cond, wr_ref[...]) + br_ref[...]

    sel = (oh[:, 0:1] * el[:, 0:E] + oh[:, 1:2] * el[:, E:2 * E]
           + oh[:, 2:3] * el[:, 2 * E:3 * E])
    mg = jnp.max(sel, axis=1, keepdims=True)
    ex = jnp.exp(sel - mg)
    w = ex / jnp.sum(ex, axis=1, keepdims=True)
    w_ref[:, 0:E] = w
    w_ref[:, E:WPAD] = jnp.zeros((T, WPAD - E), dtype=jnp.float32)

    acc = oh
    shift = 1
    while shift < T:
        acc = acc + jnp.concatenate(
            [jnp.zeros((shift, G), dtype=jnp.float32), acc[:T - shift, :]],
            axis=0)
        shift *= 2
    cs_excl = acc - oh
    counts = acc[T - 1:T, :]
    offe = jnp.concatenate(
        [jnp.zeros((1, 1), dtype=jnp.float32),
         counts[:, 0:1],
         counts[:, 0:1] + counts[:, 1:2]], axis=1)
    rank_f = jnp.sum(oh * (cs_excl + offe), axis=1, keepdims=True)
    rank_i = rank_f.astype(jnp.int32)
    rank_ref[...] = rank_i

    t_col = lax.broadcasted_iota(
        jnp.int32, (T, 1), 0).astype(jnp.float32)
    for ci in range(T // 256):
        icol = 256 * ci + lax.broadcasted_iota(jnp.int32, (1, 256), 1)
        a = (rank_i == icol).astype(jnp.float32)
        perm_ref[ci:ci + 1, :] = jnp.sum(
            a * t_col, axis=0, keepdims=True).astype(jnp.int32)

    offlo = offe
    offhi = jnp.concatenate(
        [offe[:, 1:G], jnp.full((1, 1), float(T), jnp.float32)], axis=1)
    ne = (counts > 0.0).astype(jnp.float32)
    bf = jnp.floor(offlo / float(BT))
    bl = jnp.floor((offhi - 1.0) / float(BT))
    bc = (bl - bf + 1.0) * ne
    z1 = jnp.zeros((1, 1), jnp.float32)
    a1 = bc[:, 0:1]
    a2 = a1 + bc[:, 1:2]
    a3 = a2 + bc[:, 2:3]
    bcum3 = jnp.concatenate([z1, a1, a2], axis=1)
    bcumhi = jnp.concatenate([a1, a2, a3], axis=1)
    ns0 = ne[:, 0:1]
    nsum = jnp.concatenate([z1, ns0, ns0 + ne[:, 1:2]], axis=1)
    S = ns0 + ne[:, 1:2] + ne[:, 2:3]
    jio = lax.broadcasted_iota(jnp.int32, (16, 1), 0).astype(jnp.float32)
    gh = ((jio >= bcum3) & (jio < bcumhi)).astype(jnp.float32)
    gidxr = lax.broadcasted_iota(jnp.int32, (1, G), 1).astype(jnp.float32)
    validj = jnp.sum(gh, axis=1, keepdims=True)
    gj = jnp.sum(gh * gidxr, axis=1, keepdims=True)
    bj = jnp.sum(gh * (bf - bcum3), axis=1, keepdims=True) + jio * validj
    segj = jnp.sum(gh * nsum, axis=1, keepdims=True)
    lastmask = (jio == (a3 - 1.0)).astype(jnp.float32)
    lastg = jnp.sum(lastmask * gj, axis=0, keepdims=True)
    lastb = jnp.sum(lastmask * bj, axis=0, keepdims=True)
    lastseg = jnp.sum(lastmask * segj, axis=0, keepdims=True)
    inv = 1.0 - validj
    gj = gj * validj + lastg * inv
    bj = bj * validj + lastb * inv
    segj = segj * validj + lastseg * inv
    segprev = jnp.concatenate([segj[0:1, :], segj[0:15, :]], axis=0)
    news = ((jio == 0.0) | (segj != segprev)).astype(jnp.float32)
    nlast = (segj == (S - 1.0)).astype(jnp.float32)
    stgt = (segj + 1.0) * (1.0 - nlast)
    ngj = jnp.sum(((nsum == stgt).astype(jnp.float32) * ne) * gidxr,
                  axis=1, keepdims=True)
    meta = jnp.concatenate(
        [gj, bj, validj, segj, news, ngj, nlast,
         jnp.zeros((16, 1), jnp.float32)], axis=1)
    meta_ref[...] = meta.astype(jnp.int32)

    offs = jnp.concatenate([offe, S, offhi, z1], axis=1)
    offs_ref[...] = offs.astype(jnp.int32)


def _route(x, Wg, bg, Wc, bc, Wr2, br2):
    return pl.pallas_call(
        _route_body,
        out_shape=[
            jax.ShapeDtypeStruct((T, WPAD), jnp.float32),
            jax.ShapeDtypeStruct((T, 1), jnp.int32),
            jax.ShapeDtypeStruct((T // 256, 256), jnp.int32),
            jax.ShapeDtypeStruct((1, 8), jnp.int32),
            jax.ShapeDtypeStruct((16, 8), jnp.int32),
        ],
    )(x, Wg, bg, Wc, bc, Wr2, br2)


@functools.lru_cache(maxsize=None)
def _sc_mesh_and_workers():
    info = plsc.get_sparse_core_info()
    nc, ns = info.num_cores, info.num_subcores
    mesh = plsc.VectorSubcoreMesh(
        core_axis_name="c", subcore_axis_name="s",
        num_cores=nc, num_subcores=ns)
    return mesh, nc, nc * ns


@functools.lru_cache(maxsize=None)
def _gather2_kernel():
    mesh, nc, nw = _sc_mesh_and_workers()
    bpw = T // nw

    @functools.partial(
        pl.kernel,
        out_type=[jax.ShapeDtypeStruct((T, D), jnp.float32),
                  jax.ShapeDtypeStruct((T, WPAD), jnp.float32)],
        mesh=mesh,
        scratch_types=[
            pltpu.VMEM((bpw,), jnp.int32),
            pltpu.VMEM((bpw, D), jnp.float32),
            pltpu.VMEM((bpw, WPAD), jnp.float32),
            pltpu.SemaphoreType.DMA,
            pltpu.SemaphoreType.DMA,
        ],
    )
    def k(x_hbm, w_hbm, idx_hbm, ox_hbm, ow_hbm, idx_v, rx_v, rw_v, sem,
          sem2):
        wid = lax.axis_index("s") * nc + lax.axis_index("c")
        base = wid * bpw
        pltpu.sync_copy(idx_hbm.at[pl.ds(base, bpw)], idx_v)
        cx = pltpu.async_copy(x_hbm.at[idx_v], rx_v, sem)
        cw = pltpu.async_copy(w_hbm.at[idx_v], rw_v, sem2)
        cx.wait()
        pltpu.sync_copy(rx_v, ox_hbm.at[pl.ds(base, bpw)])
        cw.wait()
        pltpu.sync_copy(rw_v, ow_hbm.at[pl.ds(base, bpw)])

    return k


@functools.lru_cache(maxsize=None)
def _gather1_kernel():
    mesh, nc, nw = _sc_mesh_and_workers()
    bpw = T // nw

    @functools.partial(
        pl.kernel,
        out_type=jax.ShapeDtypeStruct((T, D), jnp.float32),
        mesh=mesh,
        scratch_types=[
            pltpu.VMEM((bpw,), jnp.int32),
            pltpu.VMEM((bpw, D), jnp.float32),
            pltpu.SemaphoreType.DMA,
        ],
    )
    def k(ys_hbm, idx_hbm, oy_hbm, idx_v, ry_v, sem):
        wid = lax.axis_index("s") * nc + lax.axis_index("c")
        base = wid * bpw
        pltpu.sync_copy(idx_hbm.at[pl.ds(base, bpw)], idx_v)
        pltpu.async_copy(ys_hbm.at[idx_v], ry_v, sem).wait()
        pltpu.sync_copy(ry_v, oy_hbm.at[pl.ds(base, bpw)])

    return k


JT = NB + G - 1


def _ffn_body(offs_ref, meta_ref, xs_ref, ws_ref, w1_hbm, b1_ref, w2_hbm,
              b2_ref, out_ref, w1buf, w2buf, sem1, sem2):
    e = pl.program_id(0)
    j = pl.program_id(1)
    gg = meta_ref[j, 0]
    bb = meta_ref[j, 1]
    valid = meta_ref[j, 2]
    sj = meta_ref[j, 3]
    news = meta_ref[j, 4]
    ngj = meta_ref[j, 5]
    nl = meta_ref[j, 6]
    S = offs_ref[3]
    pos = e * S + sj
    p = lax.rem(pos, 2)

    @pl.when(news == 1)
    def _stage():
        @pl.when(pos == 0)
        def _prologue():
            g0 = meta_ref[0, 0]
            pltpu.make_async_copy(w1_hbm.at[g0, 0], w1buf.at[0],
                                  sem1.at[0]).start()
            pltpu.make_async_copy(w2_hbm.at[g0, 0], w2buf.at[0],
                                  sem2.at[0]).start()
        pltpu.make_async_copy(w1_hbm.at[gg, e], w1buf.at[p], sem1.at[p]).wait()
        pltpu.make_async_copy(w2_hbm.at[gg, e], w2buf.at[p], sem2.at[p]).wait()

        @pl.when(pos + 1 < E * S)
        def _next():
            en = e + nl
            pn = lax.rem(pos + 1, 2)
            pltpu.make_async_copy(w1_hbm.at[ngj, en], w1buf.at[pn],
                                  sem1.at[pn]).start()
            pltpu.make_async_copy(w2_hbm.at[ngj, en], w2buf.at[pn],
                                  sem2.at[pn]).start()

    @pl.when((e == 0) & (j == 0))
    def _init():
        out_ref[...] = xs_ref[...]

    start = offs_ref[gg]
    end = offs_ref[4 + gg]
    row0 = bb * BT

    @pl.when(valid == 1)
    def _work():
        xb = xs_ref[pl.ds(row0, BT), :]
        onehot = (lax.broadcasted_iota(jnp.int32, (WPAD, 1), 0)
                  == e).astype(jnp.float32)
        wcol = jnp.dot(ws_ref[pl.ds(row0, BT), :], onehot)
        rid = row0 + lax.broadcasted_iota(jnp.int32, (BT, 1), 0)
        inseg = (rid >= start) & (rid < end)
        wm = jnp.where(inseg, wcol, 0.0)
        h = jax.nn.gelu(jnp.dot(xb, w1buf[p]) + b1_ref[0, 0])
        contrib = jnp.dot(wm * h, w2buf[p])
        out_ref[pl.ds(row0, BT), :] += contrib + wm * b2_ref[0, 0]


def _ffn(offs, meta, xs, ws, W1, b1r, W2, b2r):
    grid_spec = pltpu.PrefetchScalarGridSpec(
        num_scalar_prefetch=2,
        grid=(E, JT),
        in_specs=[
            pl.BlockSpec((T, D), lambda e, j, offs, meta: (0, 0)),
            pl.BlockSpec((T, WPAD), lambda e, j, offs, meta: (0, 0)),
            pl.BlockSpec(memory_space=pl.ANY),
            pl.BlockSpec((1, 1, 1, FF),
                         lambda e, j, offs, meta: (meta[j, 0], e, 0, 0)),
            pl.BlockSpec(memory_space=pl.ANY),
            pl.BlockSpec((1, 1, 1, D),
                         lambda e, j, offs, meta: (meta[j, 0], e, 0, 0)),
        ],
        out_specs=pl.BlockSpec((T, D), lambda e, j, offs, meta: (0, 0)),
        scratch_shapes=[
            pltpu.VMEM((2, D, FF), jnp.float32),
            pltpu.VMEM((2, FF, D), jnp.float32),
            pltpu.SemaphoreType.DMA((2,)),
            pltpu.SemaphoreType.DMA((2,)),
        ],
    )
    return pl.pallas_call(
        _ffn_body,
        grid_spec=grid_spec,
        out_shape=jax.ShapeDtypeStruct((T, D), jnp.float32),
        compiler_params=pltpu.CompilerParams(
            dimension_semantics=("arbitrary", "arbitrary")),
    )(offs, meta, xs, ws, W1, b1r, W2, b2r)


def kernel(x, Wg, bg, Wc, bc, Wr, br, W1, b1, W2, b2):
    Wr2 = jnp.transpose(Wr, (1, 0, 2)).reshape(D, G * E)
    br2 = br.reshape(1, G * E)
    wpad, rank2, perm2, offs2, meta = _route(
        x, Wg, bg.reshape(1, G), Wc, bc.reshape(1, D), Wr2, br2)
    rank = rank2.reshape(T)
    perm = perm2.reshape(T)
    offs = offs2.reshape(8)

    xs, ws = _gather2_kernel()(x, wpad, perm)
    ys = _ffn(offs, meta, xs, ws,
              W1, b1.reshape(G, E, 1, FF), W2, b2.reshape(G, E, 1, D))
    return _gather1_kernel()(ys, rank)
